# Initial kernel scaffold; baseline (speedup 1.0000x reference)
import functools

import jax
import jax.numpy as jnp
from jax import lax
from jax.experimental import pallas as pl
from jax.experimental.pallas import tpu as pltpu
from jax.experimental.pallas import tpu_sc as plsc

LANES = 16


@functools.lru_cache(maxsize=None)
def _build(N, S, D, V, P, off, chunk):
    info = plsc.get_sparse_core_info()
    NC, NS = info.num_cores, info.num_subcores
    NW = NC * NS
    assert N % NW == 0
    b_per_w = N // NW
    assert b_per_w % chunk == 0
    n_chunks = b_per_w // chunk
    assert S % b_per_w == 0
    d_vecs = D // LANES

    mesh = plsc.VectorSubcoreMesh(core_axis_name="c", subcore_axis_name="s")

    @functools.partial(
        pl.kernel,
        mesh=mesh,
        out_type=jax.ShapeDtypeStruct((N, D), jnp.float32),
        scratch_types=[
            pltpu.VMEM((b_per_w,), jnp.int32),
            pltpu.VMEM((chunk, D), jnp.float32),
            pltpu.VMEM((chunk, D), jnp.float32),
            pltpu.SemaphoreType.DMA,
        ],
    )
    def emb(ids_hbm, tab_hbm, pos_hbm, out_hbm, idx_v, tok_v, pos_v, sem):
        wid = lax.axis_index("s") * NC + lax.axis_index("c")
        base = wid * b_per_w
        pos_base = lax.rem(base, S) + off
        pltpu.sync_copy(ids_hbm.at[pl.ds(base, b_per_w)], idx_v)
        for g in range(n_chunks):
            gather = pltpu.async_copy(
                tab_hbm.at[idx_v.at[pl.ds(g * chunk, chunk)]], tok_v, sem)
            pltpu.sync_copy(
                pos_hbm.at[pl.ds(pos_base + g * chunk, chunk)], pos_v)
            gather.wait()

            def row_add(r, _):
                for j in range(d_vecs):
                    sl = pl.ds(j * LANES, LANES)
                    tok_v[r, sl] = tok_v[r, sl] + pos_v[r, sl]
                return 0

            lax.fori_loop(0, chunk, row_add, 0)
            pltpu.sync_copy(tok_v, out_hbm.at[pl.ds(base + g * chunk, chunk)])

    return emb


def kernel(token_ids, seq_length, token_embeddings, position_embeddings):
    B, S = token_ids.shape
    V, D = token_embeddings.shape
    P = position_embeddings.shape[0]
    N = B * S
    off = int(seq_length) - S
    ids = token_ids.reshape(N).astype(jnp.int32)
    emb = _build(N, S, D, V, P, off, chunk=64)
    out = emb(ids, token_embeddings, position_embeddings)
    return out.reshape(B, S, D)


# speedup vs baseline: 1.0221x; 1.0221x over previous
import functools

import jax
import jax.numpy as jnp
from jax import lax
from jax.experimental import pallas as pl
from jax.experimental.pallas import tpu as pltpu
from jax.experimental.pallas import tpu_sc as plsc

LANES = 16


@functools.lru_cache(maxsize=None)
def _build(N, S, D, V, P, chunk):
    info = plsc.get_sparse_core_info()
    NC, NS = info.num_cores, info.num_subcores
    NW = NC * NS
    assert N % NW == 0
    b_per_w = N // NW
    assert b_per_w % chunk == 0
    n_chunks = b_per_w // chunk
    assert S % b_per_w == 0
    d_vecs = D // LANES

    mesh = plsc.VectorSubcoreMesh(core_axis_name="c", subcore_axis_name="s")

    @functools.partial(
        pl.kernel,
        mesh=mesh,
        out_type=jax.ShapeDtypeStruct((N, D), jnp.float32),
        scratch_types=[
            pltpu.VMEM((b_per_w,), jnp.int32),
            pltpu.VMEM((b_per_w,), jnp.int32),
            pltpu.VMEM((chunk, D), jnp.float32),
            pltpu.VMEM((chunk, D), jnp.float32),
            pltpu.SemaphoreType.DMA,
            pltpu.SemaphoreType.DMA,
        ],
    )
    def emb(ids_hbm, tab_hbm, pos_hbm, pid_hbm, out_hbm,
            idx_v, pid_v, tok_v, pos_v, sem, sem2):
        wid = lax.axis_index("s") * NC + lax.axis_index("c")
        base = pl.multiple_of(wid * b_per_w, b_per_w)
        pltpu.sync_copy(ids_hbm.at[pl.ds(base, b_per_w)], idx_v)
        pltpu.sync_copy(pid_hbm.at[pl.ds(base, b_per_w)], pid_v)
        for g in range(n_chunks):
            gather = pltpu.async_copy(
                tab_hbm.at[idx_v.at[pl.ds(g * chunk, chunk)]], tok_v, sem)
            pgather = pltpu.async_copy(
                pos_hbm.at[pid_v.at[pl.ds(g * chunk, chunk)]], pos_v, sem2)
            gather.wait()
            pgather.wait()

            def row_add(r, _):
                for j in range(d_vecs):
                    sl = pl.ds(j * LANES, LANES)
                    tok_v[r, sl] = tok_v[r, sl] + pos_v[r, sl]
                return 0

            lax.fori_loop(0, chunk, row_add, 0)
            pltpu.sync_copy(
                tok_v,
                out_hbm.at[pl.ds(pl.multiple_of(base + g * chunk, 8), chunk)])

    return emb


def kernel(token_ids, seq_length, token_embeddings, position_embeddings):
    B, S = token_ids.shape
    V, D = token_embeddings.shape
    P = position_embeddings.shape[0]
    N = B * S
    off = jnp.asarray(seq_length, jnp.int32) - S
    pos_ids = jnp.tile(jnp.arange(S, dtype=jnp.int32) + off, B)
    ids = token_ids.reshape(N).astype(jnp.int32)
    emb = _build(N, S, D, V, P, chunk=64)
    out = emb(ids, token_embeddings, position_embeddings, pos_ids)
    return out.reshape(B, S, D)


# speedup vs baseline: 1.1621x; 1.1370x over previous
import functools

import jax
import jax.numpy as jnp
from jax import lax
from jax.experimental import pallas as pl
from jax.experimental.pallas import tpu as pltpu
from jax.experimental.pallas import tpu_sc as plsc

LANES = 16


@functools.lru_cache(maxsize=None)
def _build(N, S, D, V, P, chunk):
    info = plsc.get_sparse_core_info()
    NC, NS = info.num_cores, info.num_subcores
    NW = NC * NS
    assert N % NW == 0
    b_per_w = N // NW
    assert b_per_w % chunk == 0
    n_chunks = b_per_w // chunk
    assert S % b_per_w == 0
    d_vecs = D // LANES
    NSLOT = 3

    mesh = plsc.VectorSubcoreMesh(core_axis_name="c", subcore_axis_name="s")

    @functools.partial(
        pl.kernel,
        mesh=mesh,
        out_type=jax.ShapeDtypeStruct((N, D), jnp.float32),
        scratch_types=(
            [pltpu.VMEM((b_per_w,), jnp.int32)] * 2
            + [pltpu.VMEM((chunk, D), jnp.float32)] * (2 * NSLOT)
            + [pltpu.SemaphoreType.DMA] * (2 * NSLOT)
        ),
    )
    def emb(ids_hbm, tab_hbm, pos_hbm, pid_hbm, out_hbm,
            idx_v, pid_v, *bufs):
        tok_v = bufs[0:NSLOT]
        pos_v = bufs[NSLOT:2 * NSLOT]
        sem_in = bufs[2 * NSLOT:3 * NSLOT]
        sem_out = bufs[3 * NSLOT:4 * NSLOT]
        wid = lax.axis_index("s") * NC + lax.axis_index("c")
        base = pl.multiple_of(wid * b_per_w, b_per_w)
        pltpu.sync_copy(ids_hbm.at[pl.ds(base, b_per_w)], idx_v)
        pltpu.sync_copy(pid_hbm.at[pl.ds(base, b_per_w)], pid_v)

        def issue_in(g):
            b = g % NSLOT
            tg = pltpu.async_copy(
                tab_hbm.at[idx_v.at[pl.ds(g * chunk, chunk)]],
                tok_v[b], sem_in[b])
            pg = pltpu.async_copy(
                pos_hbm.at[pid_v.at[pl.ds(g * chunk, chunk)]],
                pos_v[b], sem_in[b])
            return (tg, pg)

        in_d = {}
        out_d = {}
        for g in range(min(2, n_chunks)):
            in_d[g] = issue_in(g)
        for g in range(n_chunks):
            b = g % NSLOT
            for d in in_d.pop(g):
                d.wait()
            if g + 2 < n_chunks:
                if g - 1 >= 0:
                    out_d.pop(g - 1).wait()
                in_d[g + 2] = issue_in(g + 2)

            def row_add(r, _, b=b):
                for j in range(d_vecs):
                    sl = pl.ds(j * LANES, LANES)
                    tok_v[b][r, sl] = tok_v[b][r, sl] + pos_v[b][r, sl]
                return 0

            lax.fori_loop(0, chunk, row_add, 0)
            out_d[g] = pltpu.async_copy(
                tok_v[b],
                out_hbm.at[pl.ds(pl.multiple_of(base + g * chunk, 8), chunk)],
                sem_out[b])
        for g in sorted(out_d):
            out_d.pop(g).wait()

    return emb


def kernel(token_ids, seq_length, token_embeddings, position_embeddings):
    B, S = token_ids.shape
    V, D = token_embeddings.shape
    P = position_embeddings.shape[0]
    N = B * S
    off = jnp.asarray(seq_length, jnp.int32) - S
    pos_ids = jnp.tile(jnp.arange(S, dtype=jnp.int32) + off, B)
    ids = token_ids.reshape(N).astype(jnp.int32)
    emb = _build(N, S, D, V, P, chunk=16)
    out = emb(ids, token_embeddings, position_embeddings, pos_ids)
    return out.reshape(B, S, D)
